# baseline (speedup 1.0000x reference)
import functools

import jax
import jax.numpy as jnp
from jax import lax
from jax.experimental import pallas as pl
from jax.experimental.pallas import tpu as pltpu
from jax.experimental.pallas import tpu_sc as plsc

LANES = 16
NBUF = 4
CHUNK = 8
GROUP = 16


def _make_sc_add(B, L, H, V):
    info = plsc.get_sparse_core_info()
    NC, NS = info.num_cores, info.num_subcores
    NW = NC * NS
    assert B % (GROUP * NW) == 0
    b_per_w = B // NW
    mesh = plsc.VectorSubcoreMesh(core_axis_name="c", subcore_axis_name="s")

    @functools.partial(
        pl.kernel,
        mesh=mesh,
        out_type=jax.ShapeDtypeStruct((B, L, H), jnp.float32),
        scratch_types=[
            pltpu.VMEM((b_per_w,), jnp.int32),
            pltpu.VMEM((2, CHUNK, H), jnp.float32),
            pltpu.VMEM((NBUF, L, H), jnp.float32),
            pltpu.SemaphoreType.DMA,
            pltpu.SemaphoreType.DMA((2,)),
            pltpu.SemaphoreType.DMA((NBUF,)),
            pltpu.SemaphoreType.DMA((NBUF,)),
        ],
    )
    def sc_add(x_hbm, table_hbm, idx_hbm, out_hbm, idx_v, embbuf, buf,
               gsem, esem, isem, osem):
        wid = lax.axis_index("s") * NC + lax.axis_index("c")
        base = wid * b_per_w

        pltpu.sync_copy(idx_hbm.at[pl.ds(base, b_per_w)], idx_v)

        def fetch_x(s, slot):
            pltpu.async_copy(x_hbm.at[base + s], buf.at[slot], isem.at[slot])

        def fetch_emb(c8, eslot):
            pltpu.async_copy(table_hbm.at[idx_v.at[pl.ds(c8, CHUNK)]],
                             embbuf.at[eslot], esem.at[eslot])

        def wait_emb(c8, eslot):
            pltpu.make_async_copy(table_hbm.at[idx_v.at[pl.ds(c8, CHUNK)]],
                                  embbuf.at[eslot], esem.at[eslot]).wait()

        fetch_x(0, 0)
        fetch_x(1, 1)
        fetch_emb(0, 0)
        fetch_emb(CHUNK, 1)

        def step(s, slot, eslot, erow):
            pltpu.make_async_copy(x_hbm.at[base + s], buf.at[slot],
                                  isem.at[slot]).wait()

            es = tuple(embbuf[eslot, erow, pl.ds(k * LANES, LANES)]
                       for k in range(H // LANES))

            def body(j, e):
                for k in range(H // LANES):
                    v = buf[slot, j, pl.ds(k * LANES, LANES)]
                    buf[slot, j, pl.ds(k * LANES, LANES)] = v + e[k]
                return e

            lax.fori_loop(0, L, body, es, unroll=2)

            pltpu.async_copy(buf.at[slot], out_hbm.at[base + s], osem.at[slot])

            nslot = (slot + 2) % NBUF

            @pl.when(s + 2 < b_per_w)
            def _():
                @pl.when(s >= 2)
                def _():
                    pltpu.make_async_copy(buf.at[nslot],
                                          out_hbm.at[base + s - 2],
                                          osem.at[nslot]).wait()

                fetch_x(s + 2, nslot)

        def outer(g):
            for b in range(GROUP):
                s = g + b
                if b == 0:
                    wait_emb(g, 0)
                if b == CHUNK:
                    wait_emb(g + CHUNK, 1)
                step(s, b % NBUF, (b // CHUNK) % 2, b % CHUNK)
                if b == CHUNK - 1:
                    @pl.when(g + GROUP < b_per_w)
                    def _():
                        fetch_emb(g + GROUP, 0)
                if b == GROUP - 1:
                    @pl.when(g + GROUP + CHUNK < b_per_w)
                    def _():
                        fetch_emb(g + GROUP + CHUNK, 1)

        pl.loop(0, b_per_w, step=GROUP)(outer)

        for t in range(NBUF):
            s = b_per_w - NBUF + t
            pltpu.make_async_copy(buf.at[s % NBUF], out_hbm.at[base + s],
                                  osem.at[s % NBUF]).wait()

    return sc_add


def kernel(x, g_id, embedding):
    B, L, H = x.shape
    V = embedding.shape[0]
    return _make_sc_add(B, L, H, V)(x, embedding, g_id.astype(jnp.int32))
